# baseline (speedup 1.0000x reference)
import functools

import jax
import jax.numpy as jnp
from jax import lax
from jax.experimental import pallas as pl
from jax.experimental.pallas import tpu as pltpu
from jax.experimental.pallas import tpu_sc as plsc

F32 = jnp.float32
I32 = jnp.int32

NC = 2
NS = 16
NW = NC * NS
GRP = 128
N_SH = 10240
NHEADS = 8


def _ln(x, w, b):
    m = x.mean(-1, keepdims=True)
    v = ((x - m) ** 2).mean(-1, keepdims=True)
    return (x - m) / jnp.sqrt(v + 1e-5) * w + b


@functools.cache
def _mk_pre(N, H):
    BLK = 1000
    grid = N // BLK

    def full(*s):
        return pl.BlockSpec(s, lambda i: (0,) * len(s))

    blk = pl.BlockSpec((BLK, H), lambda i: (i, 0))

    def body(xs_ref, xd_ref, wq, bq, wk, wv, bv, lsw, lsb, ldw, ldb,
             q_ref, kv_ref):
        xs = _ln(xs_ref[...], lsw[...], lsb[...])
        xd = _ln(xd_ref[...], ldw[...], ldb[...])
        q_ref[...] = xd @ wq[...] + bq[...]
        kv_ref[...] = jnp.concatenate(
            [xs @ wk[...], xs @ wv[...] + bv[...]],
            axis=-1).astype(jnp.bfloat16)

    return pl.pallas_call(
        body,
        grid=(grid,),
        in_specs=[blk, blk, full(H, H), full(1, H), full(H, H), full(H, H),
                  full(1, H), full(1, H), full(1, H), full(1, H), full(1, H)],
        out_specs=[blk, pl.BlockSpec((BLK, 2 * H), lambda i: (i, 0))],
        out_shape=[jax.ShapeDtypeStruct((N, H), F32),
                   jax.ShapeDtypeStruct((N, 2 * H), jnp.bfloat16)],
    )


@functools.cache
def _mk_gather(N, Ep, Dq, Dkv):
    n_per_w = Ep // NW
    n_grp = n_per_w // GRP
    n_pair = n_grp // 2
    mesh = plsc.VectorSubcoreMesh(core_axis_name="c", subcore_axis_name="s")

    @functools.partial(
        pl.kernel,
        out_type=[jax.ShapeDtypeStruct((Ep, Dq), F32),
                  jax.ShapeDtypeStruct((Ep, Dq), I32)],
        mesh=mesh,
        scratch_types=[
            pltpu.VMEM((n_grp, GRP), I32),
            pltpu.VMEM((n_grp, GRP), I32),
            pltpu.VMEM((GRP, Dq), F32),
            pltpu.VMEM((GRP, Dq), F32),
            pltpu.VMEM((GRP, Dq), I32),
            pltpu.VMEM((GRP, Dq), I32),
            pltpu.SemaphoreType.DMA,
            pltpu.SemaphoreType.DMA,
            pltpu.SemaphoreType.DMA,
            pltpu.SemaphoreType.DMA,
        ],
    )
    def k(qtbl, kvtbl, dsti, srci, qe, kve, di_all, si_all,
          q_v0, q_v1, kv_v0, kv_v1, gs0, gs1, ws0, ws1):
        wid = lax.axis_index("s") * NC + lax.axis_index("c")
        base = wid * n_per_w
        q_v = (q_v0, q_v1)
        kv_v = (kv_v0, kv_v1)
        gs = (gs0, gs1)
        ws = (ws0, ws1)
        pltpu.sync_copy(dsti.at[wid], di_all)
        pltpu.sync_copy(srci.at[wid], si_all)

        def fire_gather(g, b):
            pltpu.async_copy(qtbl.at[di_all.at[g]], q_v[b], gs[b])
            pltpu.async_copy(kvtbl.at[si_all.at[g]], kv_v[b], gs[b])

        def wait_gather(b):
            pltpu.make_async_copy(qtbl.at[di_all.at[0]], q_v[b], gs[b]).wait()
            pltpu.make_async_copy(kvtbl.at[si_all.at[0]], kv_v[b], gs[b]).wait()

        def fire_wb(g, b):
            off = base + g * GRP
            pltpu.async_copy(q_v[b], qe.at[pl.ds(off, GRP)], ws[b])
            pltpu.async_copy(kv_v[b], kve.at[pl.ds(off, GRP)], ws[b])

        def wait_wb(b):
            pltpu.make_async_copy(q_v[b], qe.at[pl.ds(0, GRP)], ws[b]).wait()
            pltpu.make_async_copy(kv_v[b], kve.at[pl.ds(0, GRP)], ws[b]).wait()

        fire_gather(0, 0)

        def body(m, carry):
            g0 = 2 * m
            fire_gather(g0 + 1, 1)
            wait_gather(0)
            fire_wb(g0, 0)
            wait_gather(1)
            fire_wb(g0 + 1, 1)
            wait_wb(0)
            if n_grp % 2 == 1:
                fire_gather(g0 + 2, 0)
            else:
                @pl.when(m + 1 < n_pair)
                def _():
                    fire_gather(g0 + 2, 0)

            wait_wb(1)
            return carry

        lax.fori_loop(0, n_pair, body, 0)
        if n_grp % 2 == 1:
            wait_gather(0)
            fire_wb(n_grp - 1, 0)
            wait_wb(0)

    return k


@functools.cache
def _mk_edge(Ep, H, has_pos, conn):
    BLK = 2048
    grid = Ep // BLK
    scale = (H // NHEADS) ** -0.5

    def full(*s):
        return pl.BlockSpec(s, lambda i: (0,) * len(s))

    def body(*refs):
        if has_pos:
            (qe_ref, kve_ref, r_ref, wkr, wvr, bvr, lrw, lrb, S, ST,
             msg_ref, ex_ref) = refs
        else:
            qe_ref, kve_ref, S, ST, msg_ref, ex_ref = refs
        kv = kve_ref[...]
        ke = kv[:, :H].astype(F32)
        ve = kv[:, H:].astype(F32)
        if has_pos:
            rr = _ln(r_ref[...], lrw[...], lrb[...])
            ke = ke + rr @ wkr[...]
            ve = ve + rr @ wvr[...] + bvr[...]
        sim8 = ((qe_ref[...] * ke) @ S[...]) * scale
        ex128 = jnp.exp(sim8) @ ST[...]
        msg_ref[...] = ve * ex128
        ex_ref[...] = ex128

    in_specs = [pl.BlockSpec((BLK, H), lambda i: (i, 0)),
                pl.BlockSpec((BLK, 2 * H), lambda i: (i, 0))]
    if has_pos:
        in_specs += [pl.BlockSpec((BLK, conn), lambda i: (i, 0)),
                     full(conn, H), full(conn, H), full(1, H),
                     full(1, conn), full(1, conn)]
    in_specs += [full(H, NHEADS), full(NHEADS, H)]

    return pl.pallas_call(
        body,
        grid=(grid,),
        in_specs=in_specs,
        out_specs=[pl.BlockSpec((BLK, H), lambda i: (i, 0)),
                   pl.BlockSpec((BLK, H), lambda i: (i, 0))],
        out_shape=[jax.ShapeDtypeStruct((Ep, H), F32),
                   jax.ShapeDtypeStruct((Ep, H), F32)],
    )


@functools.cache
def _mk_scatter(Ep, H):
    n_per_w = Ep // NW
    n_grp = n_per_w // GRP
    n_pair = n_grp // 2
    rows_per_tile = N_SH // NS
    mesh = plsc.VectorSubcoreMesh(core_axis_name="c", subcore_axis_name="s")

    @functools.partial(
        pl.kernel,
        out_type=[jax.ShapeDtypeStruct((NC, N_SH, H), F32),
                  jax.ShapeDtypeStruct((NC, N_SH, H), F32)],
        mesh=mesh,
        scratch_types=[
            pltpu.VMEM((n_grp, GRP), I32),
            pltpu.VMEM((GRP, H), F32),
            pltpu.VMEM((GRP, H), F32),
            pltpu.VMEM_SHARED((N_SH, H), F32),
            pltpu.SemaphoreType.DMA,
            pltpu.SemaphoreType.DMA,
            pltpu.SemaphoreType.DMA,
            pltpu.SemaphoreType.DMA,
        ],
    )
    def k(msg, ex, dsti, zrows, agg2, den2, di_all, m_v0, m_v1, shacc,
          ls0, ls1, ss0, ss1):
        cid = lax.axis_index("c")
        sid = lax.axis_index("s")
        wid = sid * NC + cid
        tile_base = sid * rows_per_tile
        m_v = (m_v0, m_v1)
        ls = (ls0, ls1)
        ss = (ss0, ss1)
        pltpu.sync_copy(dsti.at[wid], di_all)

        def one_pass(edge_hbm, out_hbm):
            pltpu.sync_copy(zrows, shacc.at[pl.ds(tile_base, rows_per_tile)])
            plsc.subcore_barrier()

            def fire_load(g, b):
                off = wid * n_per_w + g * GRP
                pltpu.async_copy(edge_hbm.at[pl.ds(off, GRP)], m_v[b], ls[b])

            def wait_load(b):
                pltpu.make_async_copy(edge_hbm.at[pl.ds(0, GRP)], m_v[b],
                                      ls[b]).wait()

            def fire_scat(g, b):
                pltpu.async_copy(m_v[b], shacc.at[di_all.at[g]], ss[b],
                                 add=True)

            def wait_scat(b):
                pltpu.make_async_copy(m_v[b], shacc.at[di_all.at[0]],
                                      ss[b]).wait()

            fire_load(0, 0)

            def body(m, carry):
                g0 = 2 * m
                fire_load(g0 + 1, 1)
                wait_load(0)
                fire_scat(g0, 0)
                wait_load(1)
                fire_scat(g0 + 1, 1)
                wait_scat(0)

                if n_grp % 2 == 1:
                    fire_load(g0 + 2, 0)
                else:
                    @pl.when(m + 1 < n_pair)
                    def _():
                        fire_load(g0 + 2, 0)

                wait_scat(1)
                return carry

            lax.fori_loop(0, n_pair, body, 0)
            if n_grp % 2 == 1:
                wait_load(0)
                fire_scat(n_grp - 1, 0)
                wait_scat(0)
            plsc.subcore_barrier()
            pltpu.sync_copy(shacc.at[pl.ds(tile_base, rows_per_tile)],
                            out_hbm.at[cid, pl.ds(tile_base, rows_per_tile)])

        one_pass(msg, agg2)
        one_pass(ex, den2)

    return k


@functools.cache
def _mk_post(N, H, FF):
    BLK = 1000
    grid = N // BLK

    def full(*s):
        return pl.BlockSpec(s, lambda i: (0,) * len(s))

    def body(a2_ref, d2_ref, xdi_ref, wg, bg, ws, bs, wo, bo, ldw, ldb,
             lpw, lpb, fw1, fb1, fw2, fb2, lprew, lpreb, lpostw, lpostb,
             out_ref):
        num = a2_ref[0] + a2_ref[1]
        den = d2_ref[0] + d2_ref[1]
        agg = num / (den + 1e-16)
        xdi = xdi_ref[...]
        xd = _ln(xdi, ldw[...], ldb[...])
        g = jax.nn.sigmoid(agg @ wg[:H] + xd @ wg[H:] + bg[...])
        s = xd @ ws[...] + bs[...]
        upd = agg + g * (s - agg)
        attn_out = upd @ wo[...] + bo[...]
        x = xdi + _ln(attn_out, lpw[...], lpb[...])
        h = _ln(x, lprew[...], lpreb[...])
        ffo = jnp.maximum(h @ fw1[...] + fb1[...], 0.0) @ fw2[...] + fb2[...]
        out_ref[...] = x + _ln(ffo, lpostw[...], lpostb[...])

    return pl.pallas_call(
        body,
        grid=(grid,),
        in_specs=[pl.BlockSpec((NC, BLK, H), lambda i: (0, i, 0)),
                  pl.BlockSpec((NC, BLK, H), lambda i: (0, i, 0)),
                  pl.BlockSpec((BLK, H), lambda i: (i, 0)),
                  full(2 * H, H), full(1, H), full(H, H), full(1, H),
                  full(H, H), full(1, H), full(1, H), full(1, H),
                  full(1, H), full(1, H), full(H, FF), full(1, FF),
                  full(FF, H), full(1, H), full(1, H), full(1, H),
                  full(1, H), full(1, H)],
        out_specs=pl.BlockSpec((BLK, H), lambda i: (i, 0)),
        out_shape=jax.ShapeDtypeStruct((N, H), F32),
    )


def _layer(p, x_src_in, x_dst_in, r, edge_index, bipartite, has_pos):
    N, H = x_src_in.shape
    E = edge_index.shape[1]
    Ep = -(-E // (NW * GRP)) * (NW * GRP)
    n_grp = Ep // (NW * GRP)
    if n_grp % 2 == 0:
        Ep += NW * GRP
        n_grp += 1
    src = edge_index[0].astype(I32)
    dst = edge_index[1].astype(I32)
    src_p = jnp.concatenate([src, jnp.zeros((Ep - E,), I32)])
    dst_p = jnp.concatenate([dst, jnp.full((Ep - E,), N, I32)])
    src3 = src_p.reshape(NW, n_grp, GRP)
    dst3 = dst_p.reshape(NW, n_grp, GRP)

    def r1(a):
        return a.reshape(1, -1)

    if bipartite:
        ldw, ldb = p['ln_dst_w'], p['ln_dst_b']
    else:
        ldw, ldb = p['ln_src_w'], p['ln_src_b']

    q_tbl, kv_tbl = _mk_pre(N, H)(
        x_src_in, x_dst_in, p['Wq'], r1(p['bq']), p['Wk'], p['Wv'],
        r1(p['bv']), r1(p['ln_src_w']), r1(p['ln_src_b']), r1(ldw), r1(ldb))

    kv_i = lax.bitcast_convert_type(kv_tbl.reshape(N, H, 2), I32)
    qe, kve_i = _mk_gather(N, Ep, H, 2 * H)(q_tbl, kv_i, dst3, src3)
    kve = lax.bitcast_convert_type(kve_i, jnp.bfloat16).reshape(Ep, 2 * H)

    D = H // NHEADS
    S = (jnp.arange(H)[:, None] // D == jnp.arange(NHEADS)[None, :]).astype(F32)
    ST = S.T
    if has_pos:
        conn = r.shape[1]
        rp = jnp.concatenate([r, jnp.zeros((Ep - E, conn), F32)])
        msg, ex = _mk_edge(Ep, H, True, conn)(
            qe, kve, rp, p['Wkr'], p['Wvr'], r1(p['bvr']),
            r1(p['ln_r_w']), r1(p['ln_r_b']), S, ST)
    else:
        msg, ex = _mk_edge(Ep, H, False, 0)(qe, kve, S, ST)

    zrows = jnp.zeros((N_SH // NS, H), F32)
    agg2, den2 = _mk_scatter(Ep, H)(msg, ex, dst3, zrows)

    FF = p['ffW1'].shape[1]
    return _mk_post(N, H, FF)(
        agg2, den2, x_dst_in, p['Wg'], r1(p['bg']), p['Ws'], r1(p['bs']),
        p['Wo'], r1(p['bo']), r1(ldw), r1(ldb),
        r1(p['ln_post_w']), r1(p['ln_post_b']),
        p['ffW1'], r1(p['ffb1']), p['ffW2'], r1(p['ffb2']),
        r1(p['ln_ffpre_w']), r1(p['ln_ffpre_b']),
        r1(p['ln_ffpost_w']), r1(p['ln_ffpost_b']))


def kernel(x_agent, x_lane, lane_conn_attr, a2a_edge_index, l2l_edge_index,
           l2a_edge_index, params):
    x_lane2 = _layer(params['l2l'], x_lane, x_lane, lane_conn_attr,
                     l2l_edge_index, bipartite=False, has_pos=True)
    x_agent2 = _layer(params['a2a'], x_agent, x_agent, None,
                      a2a_edge_index, bipartite=False, has_pos=False)
    x_agent3 = _layer(params['l2a'], x_lane2, x_agent2, None,
                      l2a_edge_index, bipartite=True, has_pos=False)
    return x_agent3, x_lane2


# speedup vs baseline: 1.9463x; 1.9463x over previous
import functools

import jax
import jax.numpy as jnp
from jax import lax
from jax.experimental import pallas as pl
from jax.experimental.pallas import tpu as pltpu
from jax.experimental.pallas import tpu_sc as plsc

F32 = jnp.float32
I32 = jnp.int32

NC = 2
NS = 16
NW = NC * NS
GRP = 128
N_SH = 10240
NHEADS = 8


def _ln(x, w, b):
    m = x.mean(-1, keepdims=True)
    v = ((x - m) ** 2).mean(-1, keepdims=True)
    return (x - m) / jnp.sqrt(v + 1e-5) * w + b


@functools.cache
def _mk_pre(N, H):
    BLK = 1000
    grid = N // BLK

    def full(*s):
        return pl.BlockSpec(s, lambda i: (0,) * len(s))

    blk = pl.BlockSpec((BLK, H), lambda i: (i, 0))

    def body(xs_ref, xd_ref, wq, bq, wk, wv, bv, lsw, lsb, ldw, ldb,
             q_ref, kv_ref):
        xs = _ln(xs_ref[...], lsw[...], lsb[...])
        xd = _ln(xd_ref[...], ldw[...], ldb[...])
        q_ref[...] = xd @ wq[...] + bq[...]
        kv_ref[...] = jnp.concatenate(
            [xs @ wk[...], xs @ wv[...] + bv[...]], axis=-1)

    return pl.pallas_call(
        body,
        grid=(grid,),
        in_specs=[blk, blk, full(H, H), full(1, H), full(H, H), full(H, H),
                  full(1, H), full(1, H), full(1, H), full(1, H), full(1, H)],
        out_specs=[blk, pl.BlockSpec((BLK, 2 * H), lambda i: (i, 0))],
        out_shape=[jax.ShapeDtypeStruct((N, H), F32),
                   jax.ShapeDtypeStruct((N, 2 * H), F32)],
    )


@functools.cache
def _mk_gather(N, Ep, Dq, Dkv):
    n_per_w = Ep // NW
    n_grp = n_per_w // GRP
    n_pair = n_grp // 2
    mesh = plsc.VectorSubcoreMesh(core_axis_name="c", subcore_axis_name="s")

    @functools.partial(
        pl.kernel,
        out_type=[jax.ShapeDtypeStruct((Ep, Dq), F32),
                  jax.ShapeDtypeStruct((Ep, Dkv), F32)],
        mesh=mesh,
        scratch_types=[
            pltpu.VMEM((n_grp, GRP), I32),
            pltpu.VMEM((n_grp, GRP), I32),
            pltpu.VMEM((GRP, Dq), F32),
            pltpu.VMEM((GRP, Dq), F32),
            pltpu.VMEM((GRP, Dkv), F32),
            pltpu.VMEM((GRP, Dkv), F32),
            pltpu.SemaphoreType.DMA,
            pltpu.SemaphoreType.DMA,
            pltpu.SemaphoreType.DMA,
            pltpu.SemaphoreType.DMA,
        ],
    )
    def k(qtbl, kvtbl, dsti, srci, qe, kve, di_all, si_all,
          q_v0, q_v1, kv_v0, kv_v1, gs0, gs1, ws0, ws1):
        wid = lax.axis_index("s") * NC + lax.axis_index("c")
        base = wid * n_per_w
        q_v = (q_v0, q_v1)
        kv_v = (kv_v0, kv_v1)
        gs = (gs0, gs1)
        ws = (ws0, ws1)
        pltpu.sync_copy(dsti.at[wid], di_all)
        pltpu.sync_copy(srci.at[wid], si_all)

        def fire_gather(g, b):
            pltpu.async_copy(qtbl.at[di_all.at[g]], q_v[b], gs[b])
            pltpu.async_copy(kvtbl.at[si_all.at[g]], kv_v[b], gs[b])

        def wait_gather(b):
            pltpu.make_async_copy(qtbl.at[di_all.at[0]], q_v[b], gs[b]).wait()
            pltpu.make_async_copy(kvtbl.at[si_all.at[0]], kv_v[b], gs[b]).wait()

        def fire_wb(g, b):
            off = base + g * GRP
            pltpu.async_copy(q_v[b], qe.at[pl.ds(off, GRP)], ws[b])
            pltpu.async_copy(kv_v[b], kve.at[pl.ds(off, GRP)], ws[b])

        def wait_wb(b):
            pltpu.make_async_copy(q_v[b], qe.at[pl.ds(0, GRP)], ws[b]).wait()
            pltpu.make_async_copy(kv_v[b], kve.at[pl.ds(0, GRP)], ws[b]).wait()

        fire_gather(0, 0)

        def body(m, carry):
            g0 = 2 * m
            fire_gather(g0 + 1, 1)
            wait_gather(0)
            fire_wb(g0, 0)
            wait_gather(1)
            fire_wb(g0 + 1, 1)
            wait_wb(0)
            if n_grp % 2 == 1:
                fire_gather(g0 + 2, 0)
            else:
                @pl.when(m + 1 < n_pair)
                def _():
                    fire_gather(g0 + 2, 0)

            wait_wb(1)
            return carry

        lax.fori_loop(0, n_pair, body, 0)
        if n_grp % 2 == 1:
            wait_gather(0)
            fire_wb(n_grp - 1, 0)
            wait_wb(0)

    return k


@functools.cache
def _mk_edge(Ep, H, has_pos, conn):
    BLK = 2048
    grid = Ep // BLK
    scale = (H // NHEADS) ** -0.5

    def full(*s):
        return pl.BlockSpec(s, lambda i: (0,) * len(s))

    def body(*refs):
        if has_pos:
            (qe_ref, kve_ref, r_ref, wkr, wvr, bvr, lrw, lrb, S, ST,
             msg_ref, ex_ref) = refs
        else:
            qe_ref, kve_ref, S, ST, msg_ref, ex_ref = refs
        kv = kve_ref[...]
        ke = kv[:, :H]
        ve = kv[:, H:]
        if has_pos:
            rr = _ln(r_ref[...], lrw[...], lrb[...])
            ke = ke + rr @ wkr[...]
            ve = ve + rr @ wvr[...] + bvr[...]
        sim8 = ((qe_ref[...] * ke) @ S[...]) * scale
        ex128 = jnp.exp(sim8) @ ST[...]
        msg_ref[...] = ve * ex128
        ex_ref[...] = ex128

    in_specs = [pl.BlockSpec((BLK, H), lambda i: (i, 0)),
                pl.BlockSpec((BLK, 2 * H), lambda i: (i, 0))]
    if has_pos:
        in_specs += [pl.BlockSpec((BLK, conn), lambda i: (i, 0)),
                     full(conn, H), full(conn, H), full(1, H),
                     full(1, conn), full(1, conn)]
    in_specs += [full(H, NHEADS), full(NHEADS, H)]

    return pl.pallas_call(
        body,
        grid=(grid,),
        in_specs=in_specs,
        out_specs=[pl.BlockSpec((BLK, H), lambda i: (i, 0)),
                   pl.BlockSpec((BLK, H), lambda i: (i, 0))],
        out_shape=[jax.ShapeDtypeStruct((Ep, H), F32),
                   jax.ShapeDtypeStruct((Ep, H), F32)],
    )


@functools.cache
def _mk_scatter(Ep, H):
    n_per_w = Ep // NW
    n_grp = n_per_w // GRP
    n_pair = n_grp // 2
    rows_per_tile = N_SH // NS
    mesh = plsc.VectorSubcoreMesh(core_axis_name="c", subcore_axis_name="s")

    @functools.partial(
        pl.kernel,
        out_type=[jax.ShapeDtypeStruct((NC, N_SH, H), F32),
                  jax.ShapeDtypeStruct((NC, N_SH, H), F32)],
        mesh=mesh,
        scratch_types=[
            pltpu.VMEM((n_grp, GRP), I32),
            pltpu.VMEM((GRP, H), F32),
            pltpu.VMEM((GRP, H), F32),
            pltpu.VMEM_SHARED((N_SH, H), F32),
            pltpu.SemaphoreType.DMA,
            pltpu.SemaphoreType.DMA,
            pltpu.SemaphoreType.DMA,
            pltpu.SemaphoreType.DMA,
        ],
    )
    def k(msg, ex, dsti, zrows, agg2, den2, di_all, m_v0, m_v1, shacc,
          ls0, ls1, ss0, ss1):
        cid = lax.axis_index("c")
        sid = lax.axis_index("s")
        wid = sid * NC + cid
        tile_base = sid * rows_per_tile
        m_v = (m_v0, m_v1)
        ls = (ls0, ls1)
        ss = (ss0, ss1)
        pltpu.sync_copy(dsti.at[wid], di_all)

        def one_pass(edge_hbm, out_hbm):
            pltpu.sync_copy(zrows, shacc.at[pl.ds(tile_base, rows_per_tile)])
            plsc.subcore_barrier()

            def fire_load(g, b):
                off = wid * n_per_w + g * GRP
                pltpu.async_copy(edge_hbm.at[pl.ds(off, GRP)], m_v[b], ls[b])

            def wait_load(b):
                pltpu.make_async_copy(edge_hbm.at[pl.ds(0, GRP)], m_v[b],
                                      ls[b]).wait()

            def fire_scat(g, b):
                pltpu.async_copy(m_v[b], shacc.at[di_all.at[g]], ss[b],
                                 add=True)

            def wait_scat(b):
                pltpu.make_async_copy(m_v[b], shacc.at[di_all.at[0]],
                                      ss[b]).wait()

            fire_load(0, 0)

            def body(m, carry):
                g0 = 2 * m
                fire_load(g0 + 1, 1)
                wait_load(0)
                fire_scat(g0, 0)
                wait_load(1)
                fire_scat(g0 + 1, 1)
                wait_scat(0)

                if n_grp % 2 == 1:
                    fire_load(g0 + 2, 0)
                else:
                    @pl.when(m + 1 < n_pair)
                    def _():
                        fire_load(g0 + 2, 0)

                wait_scat(1)
                return carry

            lax.fori_loop(0, n_pair, body, 0)
            if n_grp % 2 == 1:
                wait_load(0)
                fire_scat(n_grp - 1, 0)
                wait_scat(0)
            plsc.subcore_barrier()
            pltpu.sync_copy(shacc.at[pl.ds(tile_base, rows_per_tile)],
                            out_hbm.at[cid, pl.ds(tile_base, rows_per_tile)])

        one_pass(msg, agg2)
        one_pass(ex, den2)

    return k


@functools.cache
def _mk_post(N, H, FF):
    BLK = 1000
    grid = N // BLK

    def full(*s):
        return pl.BlockSpec(s, lambda i: (0,) * len(s))

    def body(a2_ref, d2_ref, xdi_ref, wg, bg, ws, bs, wo, bo, ldw, ldb,
             lpw, lpb, fw1, fb1, fw2, fb2, lprew, lpreb, lpostw, lpostb,
             out_ref):
        num = a2_ref[0] + a2_ref[1]
        den = d2_ref[0] + d2_ref[1]
        agg = num / (den + 1e-16)
        xdi = xdi_ref[...]
        xd = _ln(xdi, ldw[...], ldb[...])
        g = jax.nn.sigmoid(agg @ wg[:H] + xd @ wg[H:] + bg[...])
        s = xd @ ws[...] + bs[...]
        upd = agg + g * (s - agg)
        attn_out = upd @ wo[...] + bo[...]
        x = xdi + _ln(attn_out, lpw[...], lpb[...])
        h = _ln(x, lprew[...], lpreb[...])
        ffo = jnp.maximum(h @ fw1[...] + fb1[...], 0.0) @ fw2[...] + fb2[...]
        out_ref[...] = x + _ln(ffo, lpostw[...], lpostb[...])

    return pl.pallas_call(
        body,
        grid=(grid,),
        in_specs=[pl.BlockSpec((NC, BLK, H), lambda i: (0, i, 0)),
                  pl.BlockSpec((NC, BLK, H), lambda i: (0, i, 0)),
                  pl.BlockSpec((BLK, H), lambda i: (i, 0)),
                  full(2 * H, H), full(1, H), full(H, H), full(1, H),
                  full(H, H), full(1, H), full(1, H), full(1, H),
                  full(1, H), full(1, H), full(H, FF), full(1, FF),
                  full(FF, H), full(1, H), full(1, H), full(1, H),
                  full(1, H), full(1, H)],
        out_specs=pl.BlockSpec((BLK, H), lambda i: (i, 0)),
        out_shape=jax.ShapeDtypeStruct((N, H), F32),
    )


def _layer(p, x_src_in, x_dst_in, r, edge_index, bipartite, has_pos):
    N, H = x_src_in.shape
    E = edge_index.shape[1]
    Ep = -(-E // (NW * GRP)) * (NW * GRP)
    n_grp = Ep // (NW * GRP)
    if n_grp % 2 == 0:
        Ep += NW * GRP
        n_grp += 1
    src = edge_index[0].astype(I32)
    dst = edge_index[1].astype(I32)
    src_p = jnp.concatenate([src, jnp.zeros((Ep - E,), I32)])
    dst_p = jnp.concatenate([dst, jnp.full((Ep - E,), N, I32)])
    src3 = src_p.reshape(NW, n_grp, GRP)
    dst3 = dst_p.reshape(NW, n_grp, GRP)

    def r1(a):
        return a.reshape(1, -1)

    if bipartite:
        ldw, ldb = p['ln_dst_w'], p['ln_dst_b']
    else:
        ldw, ldb = p['ln_src_w'], p['ln_src_b']

    q_tbl, kv_tbl = _mk_pre(N, H)(
        x_src_in, x_dst_in, p['Wq'], r1(p['bq']), p['Wk'], p['Wv'],
        r1(p['bv']), r1(p['ln_src_w']), r1(p['ln_src_b']), r1(ldw), r1(ldb))

    qe, kve = _mk_gather(N, Ep, H, 2 * H)(q_tbl, kv_tbl, dst3, src3)

    D = H // NHEADS
    S = (jnp.arange(H)[:, None] // D == jnp.arange(NHEADS)[None, :]).astype(F32)
    ST = S.T
    if has_pos:
        conn = r.shape[1]
        rp = jnp.concatenate([r, jnp.zeros((Ep - E, conn), F32)])
        msg, ex = _mk_edge(Ep, H, True, conn)(
            qe, kve, rp, p['Wkr'], p['Wvr'], r1(p['bvr']),
            r1(p['ln_r_w']), r1(p['ln_r_b']), S, ST)
    else:
        msg, ex = _mk_edge(Ep, H, False, 0)(qe, kve, S, ST)

    zrows = jnp.zeros((N_SH // NS, H), F32)
    agg2, den2 = _mk_scatter(Ep, H)(msg, ex, dst3, zrows)

    FF = p['ffW1'].shape[1]
    return _mk_post(N, H, FF)(
        agg2, den2, x_dst_in, p['Wg'], r1(p['bg']), p['Ws'], r1(p['bs']),
        p['Wo'], r1(p['bo']), r1(ldw), r1(ldb),
        r1(p['ln_post_w']), r1(p['ln_post_b']),
        p['ffW1'], r1(p['ffb1']), p['ffW2'], r1(p['ffb2']),
        r1(p['ln_ffpre_w']), r1(p['ln_ffpre_b']),
        r1(p['ln_ffpost_w']), r1(p['ln_ffpost_b']))


def kernel(x_agent, x_lane, lane_conn_attr, a2a_edge_index, l2l_edge_index,
           l2a_edge_index, params):
    x_lane2 = _layer(params['l2l'], x_lane, x_lane, lane_conn_attr,
                     l2l_edge_index, bipartite=False, has_pos=True)
    x_agent2 = _layer(params['a2a'], x_agent, x_agent, None,
                      a2a_edge_index, bipartite=False, has_pos=False)
    x_agent3 = _layer(params['l2a'], x_lane2, x_agent2, None,
                      l2a_edge_index, bipartite=True, has_pos=False)
    return x_agent3, x_lane2
